# Initial kernel scaffold; baseline (speedup 1.0000x reference)
import jax
import jax.numpy as jnp
from jax.experimental import pallas as pl


def kernel(x, pos, atom_cat, atom_dist, edge_index, edge_attr, params):
    raise NotImplementedError("write your pallas kernel here")


# speedup vs baseline: 4.0287x; 4.0287x over previous
import functools

import jax
import jax.numpy as jnp
from jax import lax
from jax.experimental import pallas as pl
from jax.experimental.pallas import tpu as pltpu
from jax.experimental.pallas import tpu_sc as plsc

N = 10000
E = 320000
K = 16
V = 64
D = 80

NC = 2
NS = 16
NW = NC * NS

CH = 128
NCHUNK = 79
EPT = NCHUNK * CH
E_PAD = NW * EPT
N_PAD = 10240
RPT = N_PAD // NS

f32 = jnp.float32
i32 = jnp.int32


def _mesh():
    return plsc.VectorSubcoreMesh(core_axis_name="c", subcore_axis_name="s")


_SC_PARAMS = pltpu.CompilerParams(use_tc_tiling_on_sc=False,
                                  needs_layout_passes=False)


def _sc_setup(posp, srcp, dstp, ones16, zeros16):
    kfn = pl.kernel(
        _sc_setup_body,
        out_type=(
            pltpu.HBM((E_PAD, 16), f32),
            pltpu.HBM((E_PAD, 16), f32),
            pltpu.HBM((NC, N_PAD, 16), f32),
        ),
        mesh=_mesh(),
        compiler_params=_SC_PARAMS,
        scratch_types=[
            pltpu.VMEM((NCHUNK, CH), i32),
            pltpu.VMEM((NCHUNK, CH), i32),
            pltpu.VMEM((CH, 16), f32),
            pltpu.VMEM((CH, 16), f32),
            pltpu.VMEM((CH, 16), f32),
            pltpu.VMEM((RPT, 16), f32),
            pltpu.VMEM_SHARED((N_PAD, 16), f32),
            pltpu.SemaphoreType.DMA,
            pltpu.SemaphoreType.DMA,
        ],
    )
    return kfn(posp, srcp, dstp, ones16, zeros16)


def _sc_setup_body(posp, srcp, dstp, ones16, zeros16,
                   ps_out, pd_out, deg_out,
                   src_v, dst_v, pbuf_a, pbuf_b, ones_v, zbuf, deg_sh,
                   sem_a, sem_b):
    c = lax.axis_index("c")
    s = lax.axis_index("s")
    wid = s * NC + c
    pltpu.sync_copy(srcp.at[wid], src_v)
    pltpu.sync_copy(dstp.at[wid], dst_v)
    pltpu.sync_copy(ones16, ones_v)
    pltpu.sync_copy(zeros16.at[pl.ds(s * RPT, RPT)], zbuf)
    pltpu.sync_copy(zbuf, deg_sh.at[pl.ds(s * RPT, RPT)])
    plsc.subcore_barrier()

    def chunk(j, carry):
        base = wid * EPT + j * CH
        ga = pltpu.async_copy(posp.at[src_v.at[j]], pbuf_a, sem_a)
        gb = pltpu.async_copy(posp.at[dst_v.at[j]], pbuf_b, sem_b)
        ga.wait()
        pltpu.sync_copy(pbuf_a, ps_out.at[pl.ds(base, CH)])
        gb.wait()
        pltpu.sync_copy(pbuf_b, pd_out.at[pl.ds(base, CH)])
        pltpu.sync_copy(ones_v, deg_sh.at[src_v.at[j]], add=True)
        return carry

    lax.fori_loop(0, NCHUNK, chunk, 0)
    plsc.subcore_barrier()
    pltpu.sync_copy(deg_sh.at[pl.ds(s * RPT, RPT)], zbuf)
    pltpu.sync_copy(zbuf, deg_out.at[c].at[pl.ds(s * RPT, RPT)])


def _sc_seg(table, gidx, sidx, zeros80):
    kfn = pl.kernel(
        _sc_seg_body,
        out_type=pltpu.HBM((NC, N_PAD, D), f32),
        mesh=_mesh(),
        compiler_params=_SC_PARAMS,
        scratch_types=[
            pltpu.VMEM((NCHUNK, CH), i32),
            pltpu.VMEM((NCHUNK, CH), i32),
            pltpu.VMEM((CH, D), f32),
            pltpu.VMEM((RPT // 4, D), f32),
            pltpu.VMEM_SHARED((N_PAD, D), f32),
            pltpu.SemaphoreType.DMA,
        ],
    )
    return kfn(table, gidx, sidx, zeros80)


def _sc_seg_body(table, gidx, sidx, zeros80, out,
                 gidx_v, sidx_v, gbuf, zbuf, acc, sem):
    c = lax.axis_index("c")
    s = lax.axis_index("s")
    wid = s * NC + c
    pltpu.sync_copy(gidx.at[wid], gidx_v)
    pltpu.sync_copy(sidx.at[wid], sidx_v)
    for part in range(4):
        r0 = s * RPT + part * (RPT // 4)
        pltpu.sync_copy(zeros80.at[pl.ds(r0, RPT // 4)], zbuf)
        pltpu.sync_copy(zbuf, acc.at[pl.ds(r0, RPT // 4)])
    plsc.subcore_barrier()

    def chunk(j, carry):
        pltpu.async_copy(table.at[gidx_v.at[j]], gbuf, sem).wait()
        pltpu.sync_copy(gbuf, acc.at[sidx_v.at[j]], add=True)
        return carry

    lax.fori_loop(0, NCHUNK, chunk, 0)
    plsc.subcore_barrier()
    for part in range(4):
        r0 = s * RPT + part * (RPT // 4)
        pltpu.sync_copy(acc.at[pl.ds(r0, RPT // 4)], zbuf)
        pltpu.sync_copy(zbuf, out.at[c].at[pl.ds(r0, RPT // 4)])


def _sc_edge(table, gidx, sidx, diste, attre, wd, wa, zeros80):
    kfn = pl.kernel(
        _sc_edge_body,
        out_type=pltpu.HBM((NC, N_PAD, D), f32),
        mesh=_mesh(),
        compiler_params=_SC_PARAMS,
        scratch_types=[
            pltpu.VMEM((NCHUNK, CH), i32),
            pltpu.VMEM((NCHUNK, CH), i32),
            pltpu.VMEM((NCHUNK, CH), f32),
            pltpu.VMEM((NCHUNK, CH), f32),
            pltpu.VMEM((D,), f32),
            pltpu.VMEM((D,), f32),
            pltpu.VMEM((CH, D), f32),
            pltpu.VMEM((CH, D), f32),
            pltpu.VMEM((RPT // 4, D), f32),
            pltpu.VMEM_SHARED((N_PAD, D), f32),
            pltpu.SemaphoreType.DMA,
        ],
    )
    return kfn(table, gidx, sidx, diste, attre, wd, wa, zeros80)


def _sc_edge_body(table, gidx, sidx, diste, attre, wd, wa, zeros80, out,
                  gidx_v, sidx_v, dist_v, attr_v, wd_v, wa_v,
                  gbuf, rbuf, zbuf, acc, sem):
    c = lax.axis_index("c")
    s = lax.axis_index("s")
    wid = s * NC + c
    pltpu.sync_copy(gidx.at[wid], gidx_v)
    pltpu.sync_copy(sidx.at[wid], sidx_v)
    pltpu.sync_copy(diste.at[wid], dist_v)
    pltpu.sync_copy(attre.at[wid], attr_v)
    pltpu.sync_copy(wd, wd_v)
    pltpu.sync_copy(wa, wa_v)
    for part in range(4):
        r0 = s * RPT + part * (RPT // 4)
        pltpu.sync_copy(zeros80.at[pl.ds(r0, RPT // 4)], zbuf)
        pltpu.sync_copy(zbuf, acc.at[pl.ds(r0, RPT // 4)])
    plsc.subcore_barrier()

    wdc = [wd_v[pl.ds(16 * f, 16)] for f in range(D // 16)]
    wac = [wa_v[pl.ds(16 * f, 16)] for f in range(D // 16)]

    def chunk(j, carry):
        pltpu.async_copy(table.at[gidx_v.at[j]], gbuf, sem).wait()
        jv = jnp.full((16,), j, i32)

        def edge(e, ecarry):
            ev = jnp.full((16,), e, i32)
            dv = plsc.load_gather(dist_v, [jv, ev])
            av = plsc.load_gather(attr_v, [jv, ev])
            for f in range(D // 16):
                gv = gbuf[e, pl.ds(16 * f, 16)]
                rbuf[e, pl.ds(16 * f, 16)] = jnp.maximum(
                    gv + dv * wdc[f] + av * wac[f], 0.0)
            return ecarry

        lax.fori_loop(0, CH, edge, 0)
        pltpu.sync_copy(rbuf, acc.at[sidx_v.at[j]], add=True)
        return carry

    lax.fori_loop(0, NCHUNK, chunk, 0)
    plsc.subcore_barrier()
    for part in range(4):
        r0 = s * RPT + part * (RPT // 4)
        pltpu.sync_copy(acc.at[pl.ds(r0, RPT // 4)], zbuf)
        pltpu.sync_copy(zbuf, out.at[c].at[pl.ds(r0, RPT // 4)])


def _full(shape):
    nd = len(shape)
    return pl.BlockSpec(shape, lambda i: (0,) * nd)


def _rows(bm, width):
    return pl.BlockSpec((bm, width), lambda i: (i, 0))


def _mm(a, w, b=None):
    out = jnp.dot(a, w, preferred_element_type=f32)
    if b is not None:
        out = out + b
    return out


BM = 1024
PRE_BM = 128
PRE_BF = PRE_BM * K


def _tc_pre_body(cat_ref, inv_ref, x_ref, emb_ref,
                 aw0, ab0, aw1, ab1, aw2, ab2, aw3, ab3,
                 bw0, bb0, bw1, bb1, bw2, bb2, bw3, bb3,
                 cw0, cb0, cw1, cb1, cw2, cb2, cw3, cb3,
                 fw, fb, gw, gb,
                 feat_ref, g_ref):
    cat = cat_ref[:]
    oh = (cat == lax.broadcasted_iota(i32, (1, V), 1)).astype(f32)
    af = _mm(oh, emb_ref[:])
    fa = jnp.concatenate([af, inv_ref[:]], axis=1)
    h = jnp.maximum(_mm(fa, aw0[:], ab0[:]), 0.0)
    h = jnp.maximum(_mm(h, aw1[:], ab1[:]), 0.0)
    h = jnp.maximum(_mm(h, aw2[:], ab2[:]), 0.0)
    h = _mm(h, aw3[:], ab3[:])
    rows = lax.broadcasted_iota(i32, (PRE_BM, PRE_BF), 0)
    cols = lax.broadcasted_iota(i32, (PRE_BM, PRE_BF), 1)
    S = (cols // K == rows).astype(f32)
    adf = _mm(S, h)
    h = jnp.maximum(_mm(adf, bw0[:], bb0[:]), 0.0)
    h = jnp.maximum(_mm(h, bw1[:], bb1[:]), 0.0)
    h = jnp.maximum(_mm(h, bw2[:], bb2[:]), 0.0)
    adf = _mm(h, bw3[:], bb3[:])
    cin = jnp.concatenate([adf, x_ref[:]], axis=1)
    h = jnp.maximum(_mm(cin, cw0[:], cb0[:]), 0.0)
    h = jnp.maximum(_mm(h, cw1[:], cb1[:]), 0.0)
    h = jnp.maximum(_mm(h, cw2[:], cb2[:]), 0.0)
    h = _mm(h, cw3[:], cb3[:])
    feat = _mm(h, fw[:], fb[:])
    feat_ref[:] = feat
    g_ref[:] = _mm(feat, gw[:], gb[:])


def _tc_pre(cat_flat, inv_flat, x_p, weights):
    grid = N_PAD // PRE_BM
    in_specs = [
        pl.BlockSpec((PRE_BF, 1), lambda i: (i, 0)),
        pl.BlockSpec((PRE_BF, 1), lambda i: (i, 0)),
        pl.BlockSpec((PRE_BM, 1), lambda i: (i, 0)),
    ] + [_full(w.shape) for w in weights]
    return pl.pallas_call(
        _tc_pre_body,
        grid=(grid,),
        in_specs=in_specs,
        out_specs=[_rows(PRE_BM, D), _rows(PRE_BM, D)],
        out_shape=[jax.ShapeDtypeStruct((N_PAD, D), f32),
                   jax.ShapeDtypeStruct((N_PAD, D), f32)],
    )(cat_flat, inv_flat, x_p, *weights)


def _tc_dist_body(ps_ref, pd_ref, out_ref):
    dv = ps_ref[:] - pd_ref[:]
    out_ref[:] = jnp.sqrt(jnp.sum(dv * dv, axis=1, keepdims=True))


def _tc_dist(ps, pd):
    bm = 2048
    return pl.pallas_call(
        _tc_dist_body,
        grid=(E_PAD // bm,),
        in_specs=[_rows(bm, 16), _rows(bm, 16)],
        out_specs=pl.BlockSpec((bm, 1), lambda i: (i, 0)),
        out_shape=jax.ShapeDtypeStruct((E_PAD, 1), f32),
    )(ps, pd)


def _tc_a_body(feat_ref, sa_ref, sb_ref, da_ref, db_ref, w2_ref, b2_ref, out_ref):
    s = sa_ref[:] + sb_ref[:]
    deg = da_ref[:, 0:1] + db_ref[:, 0:1]
    out_ref[:] = feat_ref[:] + _mm(s, w2_ref[:]) + deg * b2_ref[:]


def _tc_a(feat, sa, sb, da, db, w2, b2):
    return pl.pallas_call(
        _tc_a_body,
        grid=(N_PAD // BM,),
        in_specs=[_rows(BM, D), _rows(BM, D), _rows(BM, D),
                  _rows(BM, 16), _rows(BM, 16), _full((D, D)), _full((1, D))],
        out_specs=_rows(BM, D),
        out_shape=jax.ShapeDtypeStruct((N_PAD, D), f32),
    )(feat, sa, sb, da, db, w2, b2)


def _tc_b_body(h_ref, aa_ref, ab_ref, wa_ref, ba_ref, wb_ref, bb_ref,
               g_ref, bt_ref, w0_ref, q_ref):
    t = h_ref[:] + aa_ref[:] + ab_ref[:]
    h1 = jnp.maximum(_mm(t, wa_ref[:], ba_ref[:]), 0.0)
    h1 = _mm(h1, wb_ref[:], bb_ref[:])
    mu = jnp.mean(h1, axis=1, keepdims=True)
    xc = h1 - mu
    var = jnp.mean(xc * xc, axis=1, keepdims=True)
    h1 = xc * lax.rsqrt(var + 1e-5) * g_ref[:] + bt_ref[:]
    h1 = jnp.maximum(h1, 0.0)
    q_ref[:] = _mm(h1, w0_ref[:])


def _tc_b(h, aa, ab, wa, ba, wb, bb, g, bt, w0):
    return pl.pallas_call(
        _tc_b_body,
        grid=(N_PAD // BM,),
        in_specs=[_rows(BM, D), _rows(BM, D), _rows(BM, D),
                  _full((D, 256)), _full((1, 256)), _full((256, 256)),
                  _full((1, 256)), _full((1, 256)), _full((1, 256)),
                  _full((256, D))],
        out_specs=_rows(BM, D),
        out_shape=jax.ShapeDtypeStruct((N_PAD, D), f32),
    )(h, aa, ab, wa, ba, wb, bb, g, bt, w0)


def _tc_c_mid_body(q_ref, aa_ref, ab_ref, b0_ref, w1_ref, b1_ref,
                   gw_ref, gb_ref, feat_ref, g_ref):
    u = jnp.maximum(q_ref[:] + aa_ref[:] + ab_ref[:] + b0_ref[:], 0.0)
    feat = _mm(u, w1_ref[:], b1_ref[:])
    feat_ref[:] = feat
    g_ref[:] = _mm(feat, gw_ref[:], gb_ref[:])


def _tc_c_mid(q, aa, ab, b0, w1, b1, gw, gb):
    return pl.pallas_call(
        _tc_c_mid_body,
        grid=(N_PAD // BM,),
        in_specs=[_rows(BM, D), _rows(BM, D), _rows(BM, D),
                  _full((1, D)), _full((D, D)), _full((1, D)),
                  _full((D, D)), _full((1, D))],
        out_specs=[_rows(BM, D), _rows(BM, D)],
        out_shape=[jax.ShapeDtypeStruct((N_PAD, D), f32),
                   jax.ShapeDtypeStruct((N_PAD, D), f32)],
    )(q, aa, ab, b0, w1, b1, gw, gb)


def _tc_c_last_body(q_ref, aa_ref, ab_ref, b0_ref, w1_ref, b1_ref,
                    sw0, sb0, sw1, sb1, sw2, sb2, sw3, sb3,
                    surf_ref, feat_ref):
    u = jnp.maximum(q_ref[:] + aa_ref[:] + ab_ref[:] + b0_ref[:], 0.0)
    feat = _mm(u, w1_ref[:], b1_ref[:])
    feat_ref[:] = feat
    h = jnp.maximum(_mm(feat, sw0[:], sb0[:]), 0.0)
    h = jnp.maximum(_mm(h, sw1[:], sb1[:]), 0.0)
    h = jnp.maximum(_mm(h, sw2[:], sb2[:]), 0.0)
    surf_ref[:] = _mm(h, sw3[:], sb3[:])


def _tc_c_last(q, aa, ab, b0, w1, b1, surfw):
    return pl.pallas_call(
        _tc_c_last_body,
        grid=(N_PAD // BM,),
        in_specs=[_rows(BM, D), _rows(BM, D), _rows(BM, D),
                  _full((1, D)), _full((D, D)), _full((1, D))]
                 + [_full(w.shape) for w in surfw],
        out_specs=[_rows(BM, D), _rows(BM, D)],
        out_shape=[jax.ShapeDtypeStruct((N_PAD, D), f32),
                   jax.ShapeDtypeStruct((N_PAD, D), f32)],
    )(q, aa, ab, b0, w1, b1, *surfw)


def kernel(x, pos, atom_cat, atom_dist, edge_index, edge_attr, params):
    npad = N_PAD - N
    x_p = jnp.concatenate([x, jnp.zeros((npad, 1), f32)], axis=0)
    cat_p = jnp.concatenate(
        [atom_cat.astype(i32), jnp.zeros((npad, K), i32)], axis=0)
    adist_p = jnp.concatenate([atom_dist, jnp.ones((npad, K), f32)], axis=0)
    cat_flat = cat_p.reshape(N_PAD * K, 1)
    inv_flat = (1.0 / adist_p).reshape(N_PAD * K, 1)
    posp = jnp.concatenate(
        [jnp.concatenate([pos, jnp.zeros((N, 13), f32)], axis=1),
         jnp.zeros((npad, 16), f32)], axis=0)

    epad = E_PAD - E
    src = edge_index[0].astype(i32)
    dst = edge_index[1].astype(i32)
    srcp = jnp.concatenate([src, jnp.full((epad,), N, i32)]).reshape(
        NW, NCHUNK, CH)
    dstp = jnp.concatenate([dst, jnp.full((epad,), N, i32)]).reshape(
        NW, NCHUNK, CH)
    attrp = jnp.concatenate([edge_attr, jnp.zeros((epad,), f32)]).reshape(
        NW, NCHUNK, CH)

    ones16 = jnp.ones((CH, 16), f32)
    zeros16 = jnp.zeros((N_PAD, 16), f32)
    zeros80 = jnp.zeros((N_PAD, D), f32)

    ps, pd, deg = _sc_setup(posp, srcp, dstp, ones16, zeros16)
    dist_e = _tc_dist(ps, pd)
    distp = dist_e.reshape(NW, NCHUNK, CH)

    p = params
    def r2(b):
        return b.reshape(1, -1)
    pre_w = [p['emb']]
    for (w, b) in p['atom_a']:
        pre_w += [w, r2(b)]
    for (w, b) in p['atom_b']:
        pre_w += [w, r2(b)]
    for (w, b) in p['chem']:
        pre_w += [w, r2(b)]
    fw, fb = p['feat_scale']
    w1_0, b1_0 = p['edge_mlps'][0][0]
    pre_w += [fw, r2(fb), w1_0[2:], r2(b1_0)]
    feat, g = _tc_pre(cat_flat, inv_flat, x_p, pre_w)

    surface = None
    for l in range(5):
        (w1, b1), (w2, b2) = p['edge_mlps'][l]
        gp = p['gins']
        wd = w1[0]
        wa = w1[1]
        s2 = _sc_edge(g, dstp, srcp, distp, attrp, wd, wa, zeros80)
        h = _tc_a(feat, s2[0], s2[1], deg[0], deg[1], w2, r2(b2))
        a1 = _sc_seg(h, srcp, dstp, zeros80)
        (wa1, ba1), (wb1, bb1) = gp[l]['mlp1']
        gam, bet = gp[l]['ln']
        (w0m, b0m), (w1m, b1m) = gp[l]['mlp2']
        q = _tc_b(h, a1[0], a1[1], wa1, r2(ba1), wb1, r2(bb1),
                  r2(gam), r2(bet), w0m)
        a2 = _sc_seg(q, srcp, dstp, zeros80)
        if l < 4:
            w1n, b1n = p['edge_mlps'][l + 1][0]
            feat, g = _tc_c_mid(q, a2[0], a2[1], r2(b0m), w1m, r2(b1m),
                                w1n[2:], r2(b1n))
        else:
            surfw = []
            for (w, b) in p['surf']:
                surfw += [w, r2(b)]
            surface, feat = _tc_c_last(q, a2[0], a2[1], r2(b0m), w1m,
                                       r2(b1m), surfw)

    return (surface[:N], feat[:N])
